# baseline (speedup 1.0000x reference)
import functools

import jax
import jax.numpy as jnp
from jax import lax
from jax.experimental import pallas as pl
from jax.experimental.pallas import tpu as pltpu
from jax.experimental.pallas import tpu_sc as plsc

_B = 16384
_D = 64
_SHARED = 16
_REST = _D - _SHARED
_CHUNK = 512


@functools.cache
def _build():
    try:
        nc = plsc.get_sparse_core_info().num_cores
    except Exception:
        nc = 2
    bpc = _B // nc
    nch = bpc // _CHUNK
    mesh = plsc.ScalarSubcoreMesh(axis_name="c")

    @functools.partial(
        pl.kernel,
        mesh=mesh,
        out_type=jax.ShapeDtypeStruct((_B, _D), jnp.float32),
        scratch_types=[
            pltpu.SMEM((_CHUNK,), jnp.int32),
            pltpu.VMEM_SHARED((bpc, _D), jnp.float32),
            pltpu.SemaphoreType.DMA,
            pltpu.SemaphoreType.DMA,
            pltpu.SemaphoreType.DMA,
        ],
    )
    def gather_kernel(x_hbm, table_hbm, shared_hbm, out_hbm,
                      idx_s, stage, sem_sh, sem_tb, sem_out):
        cid = lax.axis_index("c")
        base = cid * bpc

        for c in range(nch):
            pltpu.sync_copy(x_hbm.at[pl.ds(base + c * _CHUNK, _CHUNK)], idx_s)

            def row(i, carry):
                r = idx_s[i]
                o = c * _CHUNK + i
                pltpu.async_copy(
                    shared_hbm, stage.at[o, pl.ds(0, _SHARED)], sem_sh
                )
                pltpu.async_copy(
                    table_hbm.at[r, pl.ds(_SHARED, _REST)],
                    stage.at[o, pl.ds(_SHARED, _REST)],
                    sem_tb,
                )
                return carry

            lax.fori_loop(0, _CHUNK, row, 0)

        def drain(i, carry):
            pltpu.make_async_copy(
                shared_hbm, stage.at[i, pl.ds(0, _SHARED)], sem_sh
            ).wait()
            pltpu.make_async_copy(
                table_hbm.at[0, pl.ds(_SHARED, _REST)],
                stage.at[i, pl.ds(_SHARED, _REST)],
                sem_tb,
            ).wait()
            return carry

        lax.fori_loop(0, bpc, drain, 0)
        pltpu.async_copy(stage, out_hbm.at[pl.ds(base, bpc)], sem_out).wait()

    return gather_kernel


def kernel(X, table, shared_embed):
    return _build()(X, table, shared_embed.reshape(_SHARED))
